# baseline (speedup 1.0000x reference)
import functools

import jax
import jax.numpy as jnp
from jax import lax
from jax.experimental import pallas as pl
from jax.experimental.pallas import tpu as pltpu
from jax.experimental.pallas import tpu_sc as plsc


def _make_gather(N, D, NC, NS):
    NW = NC * NS
    n_per_w = N // NW
    chunk = min(n_per_w, 512)
    n_chunks = n_per_w // chunk
    mesh = plsc.VectorSubcoreMesh(core_axis_name="c", subcore_axis_name="s")

    @functools.partial(
        pl.kernel,
        mesh=mesh,
        out_type=(
            jax.ShapeDtypeStruct((N, D), jnp.float32),
            jax.ShapeDtypeStruct((N, D), jnp.float32),
        ),
        scratch_types=[
            pltpu.VMEM((n_per_w,), jnp.int32),
            pltpu.VMEM((chunk, D), jnp.float32),
            pltpu.SemaphoreType.DMA,
        ],
        compiler_params=pltpu.CompilerParams(use_tc_tiling_on_sc=False),
    )
    def gather_k(cos_hbm, sin_hbm, idx_hbm, cos_out, sin_out, idx_v, rows_v, sem):
        wid = lax.axis_index("s") * NC + lax.axis_index("c")
        base = wid * n_per_w
        pltpu.sync_copy(idx_hbm.at[pl.ds(base, n_per_w)], idx_v)
        for c in range(n_chunks):
            pltpu.async_copy(
                cos_hbm.at[idx_v.at[pl.ds(c * chunk, chunk)]], rows_v, sem
            ).wait()
            pltpu.sync_copy(rows_v, cos_out.at[pl.ds(base + c * chunk, chunk)])
            pltpu.async_copy(
                sin_hbm.at[idx_v.at[pl.ds(c * chunk, chunk)]], rows_v, sem
            ).wait()
            pltpu.sync_copy(rows_v, sin_out.at[pl.ds(base + c * chunk, chunk)])

    return gather_k


def kernel(x, position_ids, cos_cached, sin_cached):
    B, S = position_ids.shape
    D = cos_cached.shape[1]
    N = B * S
    info = plsc.get_sparse_core_info()
    gather_k = _make_gather(N, D, info.num_cores, info.num_subcores)
    idx = position_ids.reshape(N)
    cos_flat, sin_flat = gather_k(cos_cached, sin_cached, idx)
    return (cos_flat.reshape(B, S, D).astype(x.dtype),
            sin_flat.reshape(B, S, D).astype(x.dtype))


# speedup vs baseline: 6.3519x; 6.3519x over previous
import functools

import jax
import jax.numpy as jnp
from jax import lax
from jax.experimental import pallas as pl
from jax.experimental.pallas import tpu as pltpu
from jax.experimental.pallas import tpu_sc as plsc

_L = 16
_T = 128


def _make_gather(B, S, V, D, NC, NS):
    NW = NC * NS
    N = B * S
    n_per_w = N // NW
    w_per_b = S // n_per_w
    n_grp = D // _L
    mesh = plsc.VectorSubcoreMesh(core_axis_name="c", subcore_axis_name="s")

    @functools.partial(
        pl.kernel,
        mesh=mesh,
        out_type=(
            jax.ShapeDtypeStruct((B, D, S), jnp.float32),
            jax.ShapeDtypeStruct((B, D, S), jnp.float32),
        ),
        scratch_types=[
            pltpu.VMEM((n_per_w,), jnp.int32),
            pltpu.SMEM((2,), jnp.int32),
            pltpu.VMEM((D, n_per_w), jnp.float32),
            pltpu.VMEM((D, _T), jnp.float32),
            pltpu.VMEM((D, _T), jnp.float32),
            pltpu.VMEM((D, _T), jnp.float32),
            pltpu.VMEM((D, _T), jnp.float32),
            pltpu.SemaphoreType.DMA,
        ],
        compiler_params=pltpu.CompilerParams(needs_layout_passes=False),
    )
    def gather_k(cos_t, sin_t, idx_hbm, cos_out, sin_out,
                 idx_v, sc_s, buf, ctab, stab, cblk, sblk, sem):
        wid = lax.axis_index("s") * NC + lax.axis_index("c")
        base = wid * n_per_w
        b = wid // w_per_b
        s0 = pl.multiple_of((wid % w_per_b) * n_per_w, _T)
        pltpu.sync_copy(idx_hbm.at[pl.ds(base, n_per_w)], idx_v)
        p0 = idx_v[pl.ds(0, _L)][0]
        iota = lax.iota(jnp.int32, _L)

        def check_body(k, ok):
            g = idx_v[pl.ds(k * _L, _L)]
            return ok & jnp.all(g == p0 + k * _L + iota)

        ok = lax.fori_loop(0, n_per_w // _L, check_body, True)
        ok = ok & (p0 % _T == 0) & (p0 >= 0) & (p0 + n_per_w <= V)

        @pl.when(ok)
        def _fast():
            p0a = pl.multiple_of(jnp.maximum(p0, 0), _T)
            pltpu.sync_copy(cos_t.at[:, pl.ds(p0a, n_per_w)], buf)
            pltpu.sync_copy(buf, cos_out.at[b, :, pl.ds(s0, n_per_w)])
            pltpu.sync_copy(sin_t.at[:, pl.ds(p0a, n_per_w)], buf)
            pltpu.sync_copy(buf, sin_out.at[b, :, pl.ds(s0, n_per_w)])

        @pl.when(jnp.logical_not(ok))
        def _general():
            grps_per_blk = _T // _L

            def grp_body(gi, cur_tile):
                v = jnp.clip(idx_v[pl.ds(gi * _L, _L)], 0, V - 1)
                cur = cur_tile
                for jj in range(_L):
                    pj = v[jj]
                    t = pl.multiple_of((pj // _T) * _T, _T)

                    @pl.when(t != cur)
                    def _fetch():
                        pltpu.sync_copy(cos_t.at[:, pl.ds(t, _T)], ctab)
                        pltpu.sync_copy(sin_t.at[:, pl.ds(t, _T)], stab)

                    cur = t
                    c = pj % _T
                    dst = (gi % grps_per_blk) * _L + jj + 0 * iota
                    src = c + 0 * iota
                    for g in range(n_grp):
                        rows = g * _L + iota
                        plsc.store_scatter(
                            cblk, [rows, dst],
                            plsc.load_gather(ctab, [rows, src]))
                        plsc.store_scatter(
                            sblk, [rows, dst],
                            plsc.load_gather(stab, [rows, src]))

                @pl.when(gi % grps_per_blk == grps_per_blk - 1)
                def _flush():
                    so = pl.multiple_of(
                        s0 + (gi // grps_per_blk) * _T, _T)
                    pltpu.sync_copy(cblk, cos_out.at[b, :, pl.ds(so, _T)])
                    pltpu.sync_copy(sblk, sin_out.at[b, :, pl.ds(so, _T)])

                return cur

            lax.fori_loop(0, n_per_w // _L, grp_body, jnp.int32(-1))

    return gather_k


def kernel(x, position_ids, cos_cached, sin_cached):
    B, S = position_ids.shape
    V, D = cos_cached.shape
    N = B * S
    info = plsc.get_sparse_core_info()
    gather_k = _make_gather(B, S, V, D, info.num_cores, info.num_subcores)
    idx = position_ids.reshape(N)
    cos_o, sin_o = gather_k(cos_cached.T, sin_cached.T, idx)
    return (jnp.swapaxes(cos_o, 1, 2).astype(x.dtype),
            jnp.swapaxes(sin_o, 1, 2).astype(x.dtype))
